# Initial kernel scaffold; baseline (speedup 1.0000x reference)
import jax
import jax.numpy as jnp
from jax.experimental import pallas as pl


def kernel(inputs, embed):
    raise NotImplementedError("write your pallas kernel here")


# speedup vs baseline: 1.1352x; 1.1352x over previous
import functools

import jax
import jax.numpy as jnp
from jax import lax
from jax.experimental import pallas as pl
from jax.experimental.pallas import tpu as pltpu
from jax.experimental.pallas import tpu_sc as plsc

_B, _T, _D = 16, 576, 256
_N = 8192
_M = _B * _T

_M_BLK = 512

_NC, _NS = 2, 16
_NW = _NC * _NS
_BPW = _M // _NW
_CH, _CS = 3, 96


_STRIPS = (2048, 2048, 2048, 2048)


def _argmin_body(x_ref, e_ref, f2_ref, e2_ref, idx_ref):
    x = x_ref[...]
    e = e_ref[...]
    f2 = f2_ref[...]
    e2 = e2_ref[...]
    mm = jnp.dot(x, e, preferred_element_type=jnp.float32)
    d = f2 - 2.0 * mm + e2
    acc_v = jnp.full((_M_BLK,), jnp.inf, jnp.float32)
    acc_i = jnp.zeros((_M_BLK,), jnp.int32)
    off = 0
    for csz in _STRIPS:
        dc = jax.lax.slice(d, (0, off), (_M_BLK, off + csz))
        mc = jnp.min(dc, axis=1)
        col = jax.lax.broadcasted_iota(jnp.int32, dc.shape, 1)
        ic = jnp.min(jnp.where(dc == mc[:, None], col + off, _N), axis=1)
        take = (mc < acc_v) | ((mc == acc_v) & (ic < acc_i))
        qmc = mc.astype(jnp.bfloat16).astype(jnp.float32)
        acc_v = jnp.where(take, qmc, acc_v)
        acc_i = jnp.where(take, ic, acc_i)
        off += csz
    idx_ref[0, 0, :] = acc_i


def _compute_indices(flat, embed, f2, e2):
    n_m = _M // _M_BLK
    idx3 = pl.pallas_call(
        _argmin_body,
        grid=(n_m,),
        in_specs=[
            pl.BlockSpec((_M_BLK, _D), lambda i: (i, 0)),
            pl.BlockSpec((_D, _N), lambda i: (0, 0)),
            pl.BlockSpec((_M_BLK, 1), lambda i: (i, 0)),
            pl.BlockSpec((1, _N), lambda i: (0, 0)),
        ],
        out_specs=pl.BlockSpec((1, 1, _M_BLK), lambda i: (i, 0, 0)),
        out_shape=jax.ShapeDtypeStruct((n_m, 1, _M_BLK), jnp.int32),
    )(flat, embed, f2, e2)
    return idx3.reshape(-1)


@functools.cache
def _make_sc_gather():
    mesh = plsc.VectorSubcoreMesh(core_axis_name="c", subcore_axis_name="s")

    @functools.partial(
        pl.kernel,
        mesh=mesh,
        out_type=jax.ShapeDtypeStruct((_NW, _CH, _CS, _D), jnp.float32),
        scratch_types=[
            pltpu.VMEM((_CH, _CS), jnp.int32),
            pltpu.VMEM((_CH, _CS, _D), jnp.float32),
            pltpu.SemaphoreType.DMA,
        ],
    )
    def _sc_gather(table_hbm, idx_hbm, out_hbm, idx_v, rows_v, sem):
        wid = lax.axis_index("s") * _NC + lax.axis_index("c")
        pltpu.sync_copy(idx_hbm.at[wid], idx_v)
        copies = [
            pltpu.async_copy(table_hbm.at[idx_v.at[j]], rows_v.at[j], sem)
            for j in range(_CH)
        ]
        for cp in copies:
            cp.wait()
        pltpu.sync_copy(rows_v, out_hbm.at[wid])

    return _sc_gather


def kernel(inputs, embed):
    flat = inputs.reshape(_M, _D)
    f2 = jnp.sum(flat ** 2, axis=1, keepdims=True)
    e2 = jnp.sum(embed ** 2, axis=0, keepdims=True)
    idx_flat = _compute_indices(flat, embed, f2, e2)
    table = embed.T
    idx_grp = idx_flat.reshape(_NW, _CH, _CS)
    gathered = _make_sc_gather()(table, idx_grp)
    quantized = gathered.reshape(_B, _T, _D)
    codes = jnp.concatenate([inputs, quantized], axis=-1)
    encoding_indices = idx_flat.reshape(_B, _T)
    return (quantized, codes, encoding_indices)


# speedup vs baseline: 1.1558x; 1.0182x over previous
import functools

import jax
import jax.numpy as jnp
from jax import lax
from jax.experimental import pallas as pl
from jax.experimental.pallas import tpu as pltpu
from jax.experimental.pallas import tpu_sc as plsc

_B, _T, _D = 16, 576, 256
_N = 8192
_M = _B * _T

_M_BLK = 512

_NC, _NS = 2, 16
_NW = _NC * _NS
_BPW = _M // _NW
_CH, _CS = 3, 96


_STRIPS = (2048, 2048, 2048, 2048)


def _argmin_body(x_ref, e_ref, f2_ref, e2_ref, idx_ref, es_ref):
    @pl.when(pl.program_id(0) == 0)
    def _init():
        es_ref[...] = (-2.0 * e_ref[...]).astype(jnp.bfloat16)

    x = x_ref[...]
    f2 = f2_ref[...]
    e2 = e2_ref[...]
    mm2 = jnp.dot(x.astype(jnp.bfloat16), es_ref[...],
                  preferred_element_type=jnp.float32)
    d = f2 + mm2 + e2
    csz = _STRIPS[0]
    col = jax.lax.broadcasted_iota(jnp.int32, (_M_BLK, csz), 1)
    acc_v = jnp.full((_M_BLK,), jnp.inf, jnp.float32)
    acc_i = jnp.zeros((_M_BLK,), jnp.int32)
    off = 0
    for csz in _STRIPS:
        dc = jax.lax.slice(d, (0, off), (_M_BLK, off + csz))
        mc = jnp.min(dc, axis=1)
        ic = jnp.min(jnp.where(dc == mc[:, None], col, csz), axis=1) + off
        take = (mc < acc_v) | ((mc == acc_v) & (ic < acc_i))
        qmc = mc.astype(jnp.bfloat16).astype(jnp.float32)
        acc_v = jnp.where(take, qmc, acc_v)
        acc_i = jnp.where(take, ic, acc_i)
        off += csz
    idx_ref[0, 0, :] = acc_i


def _compute_indices(flat, embed, f2, e2):
    n_m = _M // _M_BLK
    idx3 = pl.pallas_call(
        _argmin_body,
        grid=(n_m,),
        in_specs=[
            pl.BlockSpec((_M_BLK, _D), lambda i: (i, 0)),
            pl.BlockSpec((_D, _N), lambda i: (0, 0)),
            pl.BlockSpec((_M_BLK, 1), lambda i: (i, 0)),
            pl.BlockSpec((1, _N), lambda i: (0, 0)),
        ],
        out_specs=pl.BlockSpec((1, 1, _M_BLK), lambda i: (i, 0, 0)),
        out_shape=jax.ShapeDtypeStruct((n_m, 1, _M_BLK), jnp.int32),
        scratch_shapes=[pltpu.VMEM((_D, _N), jnp.bfloat16)],
    )(flat, embed, f2, e2)
    return idx3.reshape(-1)


@functools.cache
def _make_sc_gather():
    mesh = plsc.VectorSubcoreMesh(core_axis_name="c", subcore_axis_name="s")

    @functools.partial(
        pl.kernel,
        mesh=mesh,
        out_type=jax.ShapeDtypeStruct((_NW, _CH, _CS, _D), jnp.float32),
        scratch_types=[
            pltpu.VMEM((_CH, _CS), jnp.int32),
            pltpu.VMEM((_CH, _CS, _D), jnp.float32),
            pltpu.SemaphoreType.DMA,
        ],
    )
    def _sc_gather(table_hbm, idx_hbm, out_hbm, idx_v, rows_v, sem):
        wid = lax.axis_index("s") * _NC + lax.axis_index("c")
        pltpu.sync_copy(idx_hbm.at[wid], idx_v)
        copies = [
            pltpu.async_copy(table_hbm.at[idx_v.at[j]], rows_v.at[j], sem)
            for j in range(_CH)
        ]
        for cp in copies:
            cp.wait()
        pltpu.sync_copy(rows_v, out_hbm.at[wid])

    return _sc_gather


def kernel(inputs, embed):
    flat = inputs.reshape(_M, _D)
    f2 = jnp.sum(flat ** 2, axis=1, keepdims=True)
    e2 = jnp.sum(embed ** 2, axis=0, keepdims=True)
    idx_flat = _compute_indices(flat, embed, f2, e2)
    table = embed.T
    idx_grp = idx_flat.reshape(_NW, _CH, _CS)
    gathered = _make_sc_gather()(table, idx_grp)
    quantized = gathered.reshape(_B, _T, _D)
    codes = jnp.concatenate([inputs, quantized], axis=-1)
    encoding_indices = idx_flat.reshape(_B, _T)
    return (quantized, codes, encoding_indices)
